# baseline (speedup 1.0000x reference)
import functools

import jax
import jax.numpy as jnp
from jax import lax
from jax.experimental import pallas as pl
from jax.experimental.pallas import tpu as pltpu
from jax.experimental.pallas import tpu_sc as plsc

H, W = 384, 640
MAX_DET = 100
IOU_TH = 0.5
SCORE_TH = 0.5
N = 20000

NT = 16
CPT = 1280
NP = NT * CPT
NCHUNK = CPT // 16
WCHUNK = 7
OUTP = 112
MROW = 28 * 28
MROWP = 896
NEG = float(jnp.finfo(jnp.float32).min)
BIG_I = 2 ** 30


def _splat_i(x):
    return jnp.full((16,), x, jnp.int32)


def _splat_f(x):
    return jnp.full((16,), x, jnp.float32)


def _body(prop_hbm, logit_hbm, delta_hbm, mask_hbm,
          boxes_hbm, scores_hbm, classes_hbm, maskout_hbm,
          prop_vm, logit_vm, delta_vm,
          x1_vm, y1_vm, x2_vm, y2_vm, area_vm, work_vm,
          wx1_vm, wy1_vm, wx2_vm, wy2_vm, war_vm,
          cand_vm, row_vm, keep_vm, bvm, svm, cvm,
          rb0, rb1, rb2, rb3, rb4, rb5, rb6, mout_vm, shared, sem):
    c = lax.axis_index("c")
    s = lax.axis_index("s")

    @pl.when(c == 0)
    def _core0():
        t = s
        base = t * CPT
        iota_i = lax.iota(jnp.int32, 16)
        neg = jnp.float32(NEG)
        zero16 = _splat_f(0.0)
        lane0 = iota_i == 0

        pltpu.sync_copy(prop_hbm.at[pl.ds(base * 4, CPT * 4)], prop_vm)
        pltpu.sync_copy(logit_hbm.at[pl.ds(base * 2, CPT * 2)], logit_vm)
        pltpu.sync_copy(delta_hbm.at[pl.ds(base * 4, CPT * 4)], delta_vm)

        for wch in range(WCHUNK):
            sl = pl.ds(16 * wch, 16)
            wx1_vm[sl] = zero16
            wy1_vm[sl] = zero16
            wx2_vm[sl] = zero16
            wy2_vm[sl] = zero16
            war_vm[sl] = zero16

        def decode_chunk(j, carry):
            bval, bidx = carry
            rows = 16 * j + iota_i
            col4 = lambda ref, cc: plsc.load_gather(ref, [rows * 4 + cc])
            col2 = lambda ref, cc: plsc.load_gather(ref, [rows * 2 + cc])
            px1 = col4(prop_vm, 0)
            py1 = col4(prop_vm, 1)
            px2 = col4(prop_vm, 2)
            py2 = col4(prop_vm, 3)
            l0 = col2(logit_vm, 0)
            l1 = col2(logit_vm, 1)
            dx = col4(delta_vm, 0)
            dy = col4(delta_vm, 1)
            dw = col4(delta_vm, 2)
            dh = col4(delta_vm, 3)
            ww = px2 - px1
            hh = py2 - py1
            pcx = dx * ww + (px1 + 0.5 * ww)
            pcy = dy * hh + (py1 + 0.5 * hh)
            pw = jnp.exp(dw) * ww
            ph = jnp.exp(dh) * hh
            x1 = jnp.clip(pcx - 0.5 * pw, 0.0, W - 1.0)
            y1 = jnp.clip(pcy - 0.5 * ph, 0.0, H - 1.0)
            x2 = jnp.clip(pcx + 0.5 * pw, 0.0, W - 1.0)
            y2 = jnp.clip(pcy + 0.5 * ph, 0.0, H - 1.0)
            area = jnp.maximum(x2 - x1, 0.0) * jnp.maximum(y2 - y1, 0.0)
            m = jnp.maximum(l0, l1)
            e0 = jnp.exp(l0 - m)
            e1 = jnp.exp(l1 - m)
            sc = e1 / (e0 + e1)
            gid = _splat_i(base) + 16 * j + iota_i
            wk = jnp.where(sc >= SCORE_TH, sc, neg)
            wk = jnp.where(gid < N, wk, neg)
            sl = pl.ds(16 * j, 16)
            x1_vm[sl] = x1
            y1_vm[sl] = y1
            x2_vm[sl] = x2
            y2_vm[sl] = y2
            area_vm[sl] = area
            work_vm[sl] = wk
            upd = wk > bval
            return jnp.maximum(bval, wk), jnp.where(upd, gid, bidx)

        def best_from(bval, bidx):
            lv = jnp.max(bval)
            li = jnp.min(jnp.where(bval == lv, bidx, jnp.int32(BIG_I)))
            loff = li - base
            gat = lambda ref: plsc.load_gather(ref, [_splat_i(loff)])[0]
            return (lv, li, gat(x1_vm), gat(y1_vm), gat(x2_vm),
                    gat(y2_vm), gat(area_vm))

        def write_row(st, parity):
            lv, li, mx1, my1, mx2, my2, mar = st
            r = zero16
            r = jnp.where(iota_i == 0, lv, r)
            r = jnp.where(iota_i == 1, li.astype(jnp.float32), r)
            r = jnp.where(iota_i == 2, mx1, r)
            r = jnp.where(iota_i == 3, my1, r)
            r = jnp.where(iota_i == 4, mx2, r)
            r = jnp.where(iota_i == 5, my2, r)
            r = jnp.where(iota_i == 6, mar, r)
            row_vm[...] = r
            pltpu.sync_copy(row_vm, shared.at[parity, pl.ds(t * 16, 16)])

        bval0, bidx0 = lax.fori_loop(
            0, NCHUNK, decode_chunk, (_splat_f(neg), _splat_i(0)))
        st0 = best_from(bval0, bidx0)
        write_row(st0, 0)
        write_row(st0, 1)

        def rescan_cond(rst):
            return jnp.logical_not(rst[0])

        def make_rescan(i):
            def rescan(rst):
                def maxscan(j, carry):
                    bval, bidx = carry
                    wk = work_vm[pl.ds(16 * j, 16)]
                    gid = _splat_i(base + 16 * j) + iota_i
                    upd = wk > bval
                    return (jnp.maximum(bval, wk),
                            jnp.where(upd, gid, bidx))

                bval, bidx = lax.fori_loop(
                    0, NCHUNK, maxscan, (_splat_f(neg), _splat_i(0)))
                lv, li, mx1, my1, mx2, my2, mar = best_from(bval, bidx)

                def valchunk(wch, acc):
                    sl = pl.ds(16 * wch, 16)
                    qx1 = jnp.maximum(wx1_vm[sl], mx1)
                    qy1 = jnp.maximum(wy1_vm[sl], my1)
                    qx2 = jnp.minimum(wx2_vm[sl], mx2)
                    qy2 = jnp.minimum(wy2_vm[sl], my2)
                    inter = (jnp.maximum(qx2 - qx1, 0.0)
                             * jnp.maximum(qy2 - qy1, 0.0))
                    iou = inter / (war_vm[sl] + mar - inter + 1e-8)
                    wid = 16 * wch + iota_i
                    hit = (iou > IOU_TH) & (wid <= i)
                    return acc | jnp.any(hit)

                supp = lax.fori_loop(0, WCHUNK, valchunk, False)
                plsc.store_scatter(work_vm, [_splat_i(li - base)],
                                   _splat_f(neg), mask=lane0 & supp)
                return (jnp.logical_not(supp), lv, li, mx1, my1, mx2,
                        my2, mar)

            return rescan

        def nms_round(i, carry):
            lv, li, mx1, my1, mx2, my2, mar, chg_prev = carry
            plsc.subcore_barrier()
            p = lax.rem(i, 2)
            pltpu.sync_copy(shared.at[p], cand_vm)
            vals = plsc.load_gather(cand_vm, [iota_i * 16])
            idxf = plsc.load_gather(cand_vm, [iota_i * 16 + 1])
            gv = jnp.max(vals)
            gidx = jnp.min(jnp.where(vals == gv,
                                     idxf.astype(jnp.int32),
                                     jnp.int32(BIG_I)))
            ok = gv >= SCORE_TH
            wbase = (gidx // CPT) * 16
            wrow = cand_vm[pl.ds(wbase, 16)]
            wx1 = wrow[2]
            wy1 = wrow[3]
            wx2 = wrow[4]
            wy2 = wrow[5]
            war = wrow[6]
            okm = lane0 & ok
            plsc.store_scatter(wx1_vm, [_splat_i(i)], _splat_f(wx1),
                               mask=okm)
            plsc.store_scatter(wy1_vm, [_splat_i(i)], _splat_f(wy1),
                               mask=okm)
            plsc.store_scatter(wx2_vm, [_splat_i(i)], _splat_f(wx2),
                               mask=okm)
            plsc.store_scatter(wy2_vm, [_splat_i(i)], _splat_f(wy2),
                               mask=okm)
            plsc.store_scatter(war_vm, [_splat_i(i)], _splat_f(war),
                               mask=okm)
            plsc.store_scatter(
                keep_vm, [_splat_i(i)],
                _splat_i(jnp.where(ok, gidx, jnp.int32(-1))), mask=lane0)

            @pl.when(t == 0)
            def _rec():
                boxrow = plsc.load_gather(
                    cand_vm, [_splat_i(wbase + 2) + iota_i])
                boxrow = jnp.where(ok, boxrow, 0.0)
                plsc.store_scatter(bvm, [_splat_i(i), iota_i], boxrow,
                                   mask=iota_i < 4)
                plsc.store_scatter(
                    svm, [_splat_i(i)],
                    _splat_f(jnp.where(ok, gv, jnp.float32(0.0))),
                    mask=lane0)
                plsc.store_scatter(
                    cvm, [_splat_i(i)],
                    _splat_i(jnp.where(ok, 1, 0).astype(jnp.int32)),
                    mask=lane0)

            sel = ok & (li == gidx)
            qx1 = jnp.maximum(_splat_f(wx1), _splat_f(mx1))
            qy1 = jnp.maximum(_splat_f(wy1), _splat_f(my1))
            qx2 = jnp.minimum(_splat_f(wx2), _splat_f(mx2))
            qy2 = jnp.minimum(_splat_f(wy2), _splat_f(my2))
            inter = (jnp.maximum(qx2 - qx1, 0.0)
                     * jnp.maximum(qy2 - qy1, 0.0))
            iou_mine = inter / (_splat_f(mar) + _splat_f(war)
                                - inter + 1e-8)
            dead = ok & (sel | (iou_mine[0] > IOU_TH))
            plsc.store_scatter(work_vm, [_splat_i(li - base)],
                               _splat_f(neg), mask=lane0 & dead)
            rst = lax.while_loop(
                rescan_cond, make_rescan(i),
                (jnp.logical_not(dead), lv, li, mx1, my1, mx2, my2, mar))
            st = rst[1:]

            @pl.when(dead | chg_prev)
            def _pub():
                write_row(st, lax.rem(i + 1, 2))

            return st + (dead,)

        lax.fori_loop(0, MAX_DET, nms_round,
                      st0 + (jnp.bool_(False),))

        ridx = t + 16 * iota_i
        rmask = ridx < MAX_DET
        kvals = plsc.load_gather(
            keep_vm, [jnp.minimum(ridx, MAX_DET - 1)])
        validv = jnp.where(rmask & (kvals >= 0),
                           jnp.float32(1.0), jnp.float32(0.0))
        safev = jnp.where(rmask, jnp.clip(kvals, 0, N - 1), 0)
        rbufs = [rb0, rb1, rb2, rb3, rb4, rb5, rb6]
        descs = []
        for k in range(7):
            descs.append(pltpu.async_copy(
                mask_hbm.at[safev[k]], rbufs[k], sem))
        for d in descs:
            d.wait()

        for k in range(7):
            r = t + 16 * k
            vmul = validv[k]

            def ext(jj, _, k=k, vmul=vmul):
                v = rbufs[k][pl.ds(16 * jj, 16)]
                mout_vm[pl.ds(16 * jj, 16)] = v * vmul
                return 0

            lax.fori_loop(0, 49, ext, 0)

            @pl.when(r < MAX_DET)
            def _wr(r=r):
                pltpu.sync_copy(mout_vm, maskout_hbm.at[r])

        @pl.when(t == 0)
        def _fin():
            pltpu.sync_copy(bvm, boxes_hbm)
            pltpu.sync_copy(svm, scores_hbm)
            pltpu.sync_copy(cvm, classes_hbm)


_mesh = plsc.VectorSubcoreMesh(
    core_axis_name="c", subcore_axis_name="s", num_cores=2, num_subcores=16)

_sc_call = functools.partial(
    pl.kernel,
    out_type=(
        jax.ShapeDtypeStruct((OUTP, 4), jnp.float32),
        jax.ShapeDtypeStruct((OUTP,), jnp.float32),
        jax.ShapeDtypeStruct((OUTP,), jnp.int32),
        jax.ShapeDtypeStruct((MAX_DET, 28 * 28), jnp.float32),
    ),
    mesh=_mesh,
    compiler_params=pltpu.CompilerParams(needs_layout_passes=False),
    scratch_types=[
        pltpu.VMEM((CPT * 4,), jnp.float32),
        pltpu.VMEM((CPT * 2,), jnp.float32),
        pltpu.VMEM((CPT * 4,), jnp.float32),
        pltpu.VMEM((CPT,), jnp.float32),
        pltpu.VMEM((CPT,), jnp.float32),
        pltpu.VMEM((CPT,), jnp.float32),
        pltpu.VMEM((CPT,), jnp.float32),
        pltpu.VMEM((CPT,), jnp.float32),
        pltpu.VMEM((CPT,), jnp.float32),
        pltpu.VMEM((OUTP,), jnp.float32),
        pltpu.VMEM((OUTP,), jnp.float32),
        pltpu.VMEM((OUTP,), jnp.float32),
        pltpu.VMEM((OUTP,), jnp.float32),
        pltpu.VMEM((OUTP,), jnp.float32),
        pltpu.VMEM((NT * 16,), jnp.float32),
        pltpu.VMEM((16,), jnp.float32),
        pltpu.VMEM((OUTP,), jnp.int32),
        pltpu.VMEM((OUTP, 4), jnp.float32),
        pltpu.VMEM((OUTP,), jnp.float32),
        pltpu.VMEM((OUTP,), jnp.int32),
        pltpu.VMEM((MROWP,), jnp.float32),
        pltpu.VMEM((MROWP,), jnp.float32),
        pltpu.VMEM((MROWP,), jnp.float32),
        pltpu.VMEM((MROWP,), jnp.float32),
        pltpu.VMEM((MROWP,), jnp.float32),
        pltpu.VMEM((MROWP,), jnp.float32),
        pltpu.VMEM((MROWP,), jnp.float32),
        pltpu.VMEM((28 * 28,), jnp.float32),
        pltpu.VMEM_SHARED((2, NT * 16), jnp.float32),
        pltpu.SemaphoreType.DMA,
    ],
)(_body)


def kernel(proposals, cls_logits, bbox_preds, mask_preds):
    pad = NP - N
    prop = jnp.pad(proposals, ((0, pad), (0, 0))).reshape(-1)
    logit = jnp.pad(cls_logits, ((0, pad), (0, 0))).reshape(-1)
    delta = jnp.pad(bbox_preds[:, 4:8], ((0, pad), (0, 0))).reshape(-1)
    masks = jnp.pad(mask_preds.reshape(N, MROW, 2)[:, :, 1],
                    ((0, 0), (0, MROWP - MROW)))
    boxes, scores, classes, masksout = _sc_call(prop, logit, delta, masks)
    return (boxes[:MAX_DET], scores[:MAX_DET], classes[:MAX_DET],
            masksout.reshape(MAX_DET, 28, 28))


# speedup vs baseline: 1.1113x; 1.1113x over previous
import functools

import jax
import jax.numpy as jnp
from jax import lax
from jax.experimental import pallas as pl
from jax.experimental.pallas import tpu as pltpu
from jax.experimental.pallas import tpu_sc as plsc

H, W = 384, 640
MAX_DET = 100
IOU_TH = 0.5
SCORE_TH = 0.5
N = 20000

NT = 16
CPT = 1280
NP = NT * CPT
NCHUNK = CPT // 16
WCHUNK = 7
OUTP = 112
MROW = 28 * 28
NEG = float(jnp.finfo(jnp.float32).min)
BIG_I = 2 ** 30


def _splat_i(x):
    return jnp.full((16,), x, jnp.int32)


def _splat_f(x):
    return jnp.full((16,), x, jnp.float32)


def _body(prop_hbm, logit_hbm, delta_hbm, mask_hbm,
          boxes_hbm, scores_hbm, classes_hbm, maskout_hbm,
          prop_vm, logit_vm, delta_vm,
          x1_vm, y1_vm, x2_vm, y2_vm, area_vm, work_vm,
          wx1_vm, wy1_vm, wx2_vm, wy2_vm, war_vm,
          cand_vm, row_vm, keep_vm, bvm, svm, cvm,
          rb0, rb1, rb2, rb3, rb4, rb5, rb6, mout_vm, shared, sem):
    c = lax.axis_index("c")
    s = lax.axis_index("s")

    @pl.when(c == 0)
    def _core0():
        t = s
        base = t * CPT
        iota_i = lax.iota(jnp.int32, 16)
        neg = jnp.float32(NEG)
        zero16 = _splat_f(0.0)
        lane0 = iota_i == 0

        pltpu.sync_copy(prop_hbm.at[pl.ds(base * 4, CPT * 4)], prop_vm)
        pltpu.sync_copy(logit_hbm.at[pl.ds(base * 2, CPT * 2)], logit_vm)
        pltpu.sync_copy(delta_hbm.at[pl.ds(base * 4, CPT * 4)], delta_vm)

        for wch in range(WCHUNK):
            sl = pl.ds(16 * wch, 16)
            wx1_vm[sl] = zero16
            wy1_vm[sl] = zero16
            wx2_vm[sl] = zero16
            wy2_vm[sl] = zero16
            war_vm[sl] = zero16

        def decode_chunk(j, carry):
            bval, bidx = carry
            rows = 16 * j + iota_i
            col4 = lambda ref, cc: plsc.load_gather(ref, [rows * 4 + cc])
            col2 = lambda ref, cc: plsc.load_gather(ref, [rows * 2 + cc])
            px1 = col4(prop_vm, 0)
            py1 = col4(prop_vm, 1)
            px2 = col4(prop_vm, 2)
            py2 = col4(prop_vm, 3)
            l0 = col2(logit_vm, 0)
            l1 = col2(logit_vm, 1)
            dx = col4(delta_vm, 0)
            dy = col4(delta_vm, 1)
            dw = col4(delta_vm, 2)
            dh = col4(delta_vm, 3)
            ww = px2 - px1
            hh = py2 - py1
            pcx = dx * ww + (px1 + 0.5 * ww)
            pcy = dy * hh + (py1 + 0.5 * hh)
            pw = jnp.exp(dw) * ww
            ph = jnp.exp(dh) * hh
            x1 = jnp.clip(pcx - 0.5 * pw, 0.0, W - 1.0)
            y1 = jnp.clip(pcy - 0.5 * ph, 0.0, H - 1.0)
            x2 = jnp.clip(pcx + 0.5 * pw, 0.0, W - 1.0)
            y2 = jnp.clip(pcy + 0.5 * ph, 0.0, H - 1.0)
            area = jnp.maximum(x2 - x1, 0.0) * jnp.maximum(y2 - y1, 0.0)
            m = jnp.maximum(l0, l1)
            e0 = jnp.exp(l0 - m)
            e1 = jnp.exp(l1 - m)
            sc = e1 / (e0 + e1)
            gid = _splat_i(base) + 16 * j + iota_i
            wk = jnp.where(sc >= SCORE_TH, sc, neg)
            wk = jnp.where(gid < N, wk, neg)
            sl = pl.ds(16 * j, 16)
            x1_vm[sl] = x1
            y1_vm[sl] = y1
            x2_vm[sl] = x2
            y2_vm[sl] = y2
            area_vm[sl] = area
            work_vm[sl] = wk
            upd = wk > bval
            return jnp.maximum(bval, wk), jnp.where(upd, gid, bidx)

        def best_from(bval, bidx):
            lv = jnp.max(bval)
            li = jnp.min(jnp.where(bval == lv, bidx, jnp.int32(BIG_I)))
            loff = li - base
            gat = lambda ref: plsc.load_gather(ref, [_splat_i(loff)])[0]
            return (lv, li, gat(x1_vm), gat(y1_vm), gat(x2_vm),
                    gat(y2_vm), gat(area_vm))

        def write_row(st, parity):
            lv, li, mx1, my1, mx2, my2, mar = st
            r = zero16
            r = jnp.where(iota_i == 0, lv, r)
            r = jnp.where(iota_i == 1, li.astype(jnp.float32), r)
            r = jnp.where(iota_i == 2, mx1, r)
            r = jnp.where(iota_i == 3, my1, r)
            r = jnp.where(iota_i == 4, mx2, r)
            r = jnp.where(iota_i == 5, my2, r)
            r = jnp.where(iota_i == 6, mar, r)
            row_vm[...] = r
            pltpu.sync_copy(row_vm, shared.at[parity, pl.ds(t * 16, 16)])

        bval0, bidx0 = lax.fori_loop(
            0, NCHUNK, decode_chunk, (_splat_f(neg), _splat_i(0)))
        st0 = best_from(bval0, bidx0)
        write_row(st0, 0)
        write_row(st0, 1)

        def rescan_cond(rst):
            return jnp.logical_not(rst[0])

        def make_rescan(i):
            def rescan(rst):
                def maxscan(j, carry):
                    bval, bidx = carry
                    wk = work_vm[pl.ds(16 * j, 16)]
                    gid = _splat_i(base + 16 * j) + iota_i
                    upd = wk > bval
                    return (jnp.maximum(bval, wk),
                            jnp.where(upd, gid, bidx))

                bval, bidx = lax.fori_loop(
                    0, NCHUNK, maxscan, (_splat_f(neg), _splat_i(0)))
                lv, li, mx1, my1, mx2, my2, mar = best_from(bval, bidx)

                def valchunk(wch, acc):
                    sl = pl.ds(16 * wch, 16)
                    qx1 = jnp.maximum(wx1_vm[sl], mx1)
                    qy1 = jnp.maximum(wy1_vm[sl], my1)
                    qx2 = jnp.minimum(wx2_vm[sl], mx2)
                    qy2 = jnp.minimum(wy2_vm[sl], my2)
                    inter = (jnp.maximum(qx2 - qx1, 0.0)
                             * jnp.maximum(qy2 - qy1, 0.0))
                    iou = inter / (war_vm[sl] + mar - inter + 1e-8)
                    wid = 16 * wch + iota_i
                    hit = (iou > IOU_TH) & (wid <= i)
                    return acc | jnp.any(hit)

                supp = lax.fori_loop(0, WCHUNK, valchunk, False)
                plsc.store_scatter(work_vm, [_splat_i(li - base)],
                                   _splat_f(neg), mask=lane0 & supp)
                return (jnp.logical_not(supp), lv, li, mx1, my1, mx2,
                        my2, mar)

            return rescan

        def nms_round(i, carry):
            lv, li, mx1, my1, mx2, my2, mar, chg_prev = carry
            plsc.subcore_barrier()
            p = lax.rem(i, 2)
            pltpu.sync_copy(shared.at[p], cand_vm)
            vals = plsc.load_gather(cand_vm, [iota_i * 16])
            idxf = plsc.load_gather(cand_vm, [iota_i * 16 + 1])
            gv = jnp.max(vals)
            gidx = jnp.min(jnp.where(vals == gv,
                                     idxf.astype(jnp.int32),
                                     jnp.int32(BIG_I)))
            ok = gv >= SCORE_TH
            wbase = (gidx // CPT) * 16
            wrow = cand_vm[pl.ds(wbase, 16)]
            wx1 = wrow[2]
            wy1 = wrow[3]
            wx2 = wrow[4]
            wy2 = wrow[5]
            war = wrow[6]
            okm = lane0 & ok
            plsc.store_scatter(wx1_vm, [_splat_i(i)], _splat_f(wx1),
                               mask=okm)
            plsc.store_scatter(wy1_vm, [_splat_i(i)], _splat_f(wy1),
                               mask=okm)
            plsc.store_scatter(wx2_vm, [_splat_i(i)], _splat_f(wx2),
                               mask=okm)
            plsc.store_scatter(wy2_vm, [_splat_i(i)], _splat_f(wy2),
                               mask=okm)
            plsc.store_scatter(war_vm, [_splat_i(i)], _splat_f(war),
                               mask=okm)
            plsc.store_scatter(
                keep_vm, [_splat_i(i)],
                _splat_i(jnp.where(ok, gidx, jnp.int32(-1))), mask=lane0)

            @pl.when(t == 0)
            def _rec():
                boxrow = plsc.load_gather(
                    cand_vm, [_splat_i(wbase + 2) + iota_i])
                boxrow = jnp.where(ok, boxrow, 0.0)
                plsc.store_scatter(bvm, [_splat_i(i), iota_i], boxrow,
                                   mask=iota_i < 4)
                plsc.store_scatter(
                    svm, [_splat_i(i)],
                    _splat_f(jnp.where(ok, gv, jnp.float32(0.0))),
                    mask=lane0)
                plsc.store_scatter(
                    cvm, [_splat_i(i)],
                    _splat_i(jnp.where(ok, 1, 0).astype(jnp.int32)),
                    mask=lane0)

            sel = ok & (li == gidx)
            qx1 = jnp.maximum(_splat_f(wx1), _splat_f(mx1))
            qy1 = jnp.maximum(_splat_f(wy1), _splat_f(my1))
            qx2 = jnp.minimum(_splat_f(wx2), _splat_f(mx2))
            qy2 = jnp.minimum(_splat_f(wy2), _splat_f(my2))
            inter = (jnp.maximum(qx2 - qx1, 0.0)
                     * jnp.maximum(qy2 - qy1, 0.0))
            iou_mine = inter / (_splat_f(mar) + _splat_f(war)
                                - inter + 1e-8)
            dead = ok & (sel | (iou_mine[0] > IOU_TH))
            plsc.store_scatter(work_vm, [_splat_i(li - base)],
                               _splat_f(neg), mask=lane0 & dead)
            rst = lax.while_loop(
                rescan_cond, make_rescan(i),
                (jnp.logical_not(dead), lv, li, mx1, my1, mx2, my2, mar))
            st = rst[1:]

            @pl.when(dead | chg_prev)
            def _pub():
                write_row(st, lax.rem(i + 1, 2))

            return st + (dead,)

        lax.fori_loop(0, MAX_DET, nms_round,
                      st0 + (jnp.bool_(False),))

        ridx = t + 16 * iota_i
        rmask = ridx < MAX_DET
        kvals = plsc.load_gather(
            keep_vm, [jnp.minimum(ridx, MAX_DET - 1)])
        validv = jnp.where(rmask & (kvals >= 0),
                           jnp.float32(1.0), jnp.float32(0.0))
        safev = jnp.where(rmask, jnp.clip(kvals, 0, N - 1), 0)
        rbufs = [rb0, rb1, rb2, rb3, rb4, rb5, rb6]
        descs = []
        for k in range(7):
            descs.append(pltpu.async_copy(
                mask_hbm.at[safev[k]], rbufs[k], sem))
        for d in descs:
            d.wait()

        for k in range(7):
            r = t + 16 * k
            vmul = validv[k]

            def ext(jj, _, k=k, vmul=vmul):
                v = rbufs[k][pl.ds(16 * jj, 16)]
                mout_vm[pl.ds(16 * jj, 16)] = v * vmul
                return 0

            lax.fori_loop(0, 49, ext, 0)

            @pl.when(r < MAX_DET)
            def _wr(r=r):
                pltpu.sync_copy(mout_vm, maskout_hbm.at[r])

        @pl.when(t == 0)
        def _fin():
            pltpu.sync_copy(bvm, boxes_hbm)
            pltpu.sync_copy(svm, scores_hbm)
            pltpu.sync_copy(cvm, classes_hbm)


_mesh = plsc.VectorSubcoreMesh(
    core_axis_name="c", subcore_axis_name="s", num_cores=2, num_subcores=16)

_sc_call = functools.partial(
    pl.kernel,
    out_type=(
        jax.ShapeDtypeStruct((OUTP, 4), jnp.float32),
        jax.ShapeDtypeStruct((OUTP,), jnp.float32),
        jax.ShapeDtypeStruct((OUTP,), jnp.int32),
        jax.ShapeDtypeStruct((MAX_DET, 28 * 28), jnp.float32),
    ),
    mesh=_mesh,
    compiler_params=pltpu.CompilerParams(needs_layout_passes=False),
    scratch_types=[
        pltpu.VMEM((CPT * 4,), jnp.float32),
        pltpu.VMEM((CPT * 2,), jnp.float32),
        pltpu.VMEM((CPT * 4,), jnp.float32),
        pltpu.VMEM((CPT,), jnp.float32),
        pltpu.VMEM((CPT,), jnp.float32),
        pltpu.VMEM((CPT,), jnp.float32),
        pltpu.VMEM((CPT,), jnp.float32),
        pltpu.VMEM((CPT,), jnp.float32),
        pltpu.VMEM((CPT,), jnp.float32),
        pltpu.VMEM((OUTP,), jnp.float32),
        pltpu.VMEM((OUTP,), jnp.float32),
        pltpu.VMEM((OUTP,), jnp.float32),
        pltpu.VMEM((OUTP,), jnp.float32),
        pltpu.VMEM((OUTP,), jnp.float32),
        pltpu.VMEM((NT * 16,), jnp.float32),
        pltpu.VMEM((16,), jnp.float32),
        pltpu.VMEM((OUTP,), jnp.int32),
        pltpu.VMEM((OUTP, 4), jnp.float32),
        pltpu.VMEM((OUTP,), jnp.float32),
        pltpu.VMEM((OUTP,), jnp.int32),
        pltpu.VMEM((MROW,), jnp.float32),
        pltpu.VMEM((MROW,), jnp.float32),
        pltpu.VMEM((MROW,), jnp.float32),
        pltpu.VMEM((MROW,), jnp.float32),
        pltpu.VMEM((MROW,), jnp.float32),
        pltpu.VMEM((MROW,), jnp.float32),
        pltpu.VMEM((MROW,), jnp.float32),
        pltpu.VMEM((28 * 28,), jnp.float32),
        pltpu.VMEM_SHARED((2, NT * 16), jnp.float32),
        pltpu.SemaphoreType.DMA,
    ],
)(_body)


def kernel(proposals, cls_logits, bbox_preds, mask_preds):
    pad = NP - N
    prop = jnp.pad(proposals.reshape(-1), (0, pad * 4))
    logit = jnp.pad(cls_logits.reshape(-1), (0, pad * 2))
    delta = jnp.pad(bbox_preds[:, 4:8].reshape(-1), (0, pad * 4))
    masks = mask_preds.reshape(N, MROW, 2)[:, :, 1]
    boxes, scores, classes, masksout = _sc_call(prop, logit, delta, masks)
    return (boxes[:MAX_DET], scores[:MAX_DET], classes[:MAX_DET],
            masksout.reshape(MAX_DET, 28, 28))
